# baseline (speedup 1.0000x reference)
import functools

import jax
import jax.numpy as jnp
from jax import lax
from jax.experimental import pallas as pl
from jax.experimental.pallas import tpu as pltpu
from jax.experimental.pallas import tpu_sc as plsc

ROLE_SUBJ = 0
ROLE_OBJ = 1
ROLE_VERB = 2
ROLE_CONJ = 3
LN_EPS = 1e-5

_NC = 2
_NS = 16
_NW = _NC * _NS


def _rotvec_body(x_ref, wrot_ref, brot_ref, role_ref, ptr_ref,
                 rv_ref, sel_ref):
    xb = x_ref[...]
    role = role_ref[...]
    rv_ref[...] = jnp.dot(xb, wrot_ref[...],
                          preferred_element_type=jnp.float32) + brot_ref[...]
    is_conj = role == ROLE_CONJ
    is_other = role > ROLE_CONJ
    sel = jnp.where(is_conj, 1, jnp.where(is_other, 2, 0))
    sel_ref[...] = sel * 16 + ptr_ref[...]


def _main_body(x_ref, ms_ref, mo_ref, mv_ref,
               wrole_ref, brole_ref, wspawn_ref, bspawn_ref,
               wsubj_ref, bsubj_ref, wobj_ref, bobj_ref,
               wvg_ref, bvg_ref, wvc_ref, bvc_ref,
               wog_ref, bog_ref, gamma_ref, beta_ref,
               role_ref, slock_ref, olock_ref,
               xout_ref, rl_ref, sp_ref, ns_ref, no_ref, nv_ref):
    xb = x_ref[...]
    role = role_ref[...]
    slock = slock_ref[...] != 0
    olock = olock_ref[...] != 0

    is_subj = role == ROLE_SUBJ
    is_obj = role == ROLE_OBJ
    is_verb = role == ROLE_VERB

    rl_ref[...] = jnp.dot(xb, wrole_ref[...],
                          preferred_element_type=jnp.float32) + brole_ref[...]
    sp_ref[...] = jnp.dot(xb, wspawn_ref[...],
                          preferred_element_type=jnp.float32) + bspawn_ref[0, 0]

    subj_vec = jnp.dot(xb, wsubj_ref[...],
                       preferred_element_type=jnp.float32) + bsubj_ref[...]
    ns_ref[...] = jnp.where(is_subj & (~slock), subj_vec, ms_ref[...])
    obj_vec = jnp.dot(xb, wobj_ref[...],
                      preferred_element_type=jnp.float32) + bobj_ref[...]
    no_ref[...] = jnp.where(is_obj & (~olock), obj_vec, mo_ref[...])

    vgate = jax.nn.sigmoid(jnp.dot(xb, wvg_ref[...],
                                   preferred_element_type=jnp.float32)
                           + bvg_ref[0, 0])
    verb_vec = jnp.dot(xb, wvc_ref[...],
                       preferred_element_type=jnp.float32) + bvc_ref[...]
    mv = mv_ref[...]
    nv_ref[...] = jnp.where(is_verb, vgate * verb_vec + (1.0 - vgate) * mv, mv)

    gate = jax.nn.sigmoid(jnp.dot(xb, wog_ref[...],
                                  preferred_element_type=jnp.float32)
                          + bog_ref[...])
    h = xb + gate * xb
    mu = jnp.mean(h, axis=-1, keepdims=True)
    var = jnp.mean((h - mu) ** 2, axis=-1, keepdims=True)
    xout_ref[...] = ((h - mu) * lax.rsqrt(var + LN_EPS) * gamma_ref[...]
                     + beta_ref[...])


def _sc_rot(B, R, d):
    RW = B // _NW
    CH = 4
    NCHUNK = RW // CH
    mesh = plsc.VectorSubcoreMesh(core_axis_name="c", subcore_axis_name="s",
                                  num_cores=_NC, num_subcores=_NS)

    @functools.partial(
        pl.kernel,
        out_type=jax.ShapeDtypeStruct((B * R, d), jnp.float32),
        mesh=mesh,
    scratch_types=[
            pltpu.VMEM((CH * R, d), jnp.float32),
            pltpu.VMEM((RW + 16,), jnp.int32),
            pltpu.SemaphoreType.DMA,
            pltpu.SemaphoreType.DMA,
        ],
    )
    def sc_rot(rot_hbm, x_hbm, rv_hbm, sel_hbm, z_hbm, out_hbm,
               buf, codev, sem_in, sem_out):
        wid = lax.axis_index("s") * _NC + lax.axis_index("c")
        base = wid * RW
        pltpu.sync_copy(sel_hbm.at[pl.ds(base, RW)], codev.at[pl.ds(0, RW)])

        def chunk(c, carry):
            row0 = base + c * CH
            pltpu.async_copy(rot_hbm.at[pl.ds(row0 * R, CH * R)], buf,
                             sem_in).wait()
            for r in range(CH):
                j = c * CH + r
                code = codev[pl.ds(j, 16)][0]
                s = code >> 4
                p = code & 15
                fr = r * R

                @pl.when(s == 1)
                def _():
                    pltpu.sync_copy(z_hbm.at[pl.ds(0, R - 1)],
                                    buf.at[pl.ds(fr + 1, R - 1)])
                    pltpu.sync_copy(x_hbm.at[pl.ds(row0 + r, 1)],
                                    buf.at[pl.ds(fr, 1)])

                @pl.when(s == 2)
                def _():
                    pltpu.sync_copy(rv_hbm.at[pl.ds(row0 + r, 1)],
                                    buf.at[pl.ds(fr + p, 1)])
            pltpu.async_copy(buf, out_hbm.at[pl.ds(row0 * R, CH * R)],
                             sem_out).wait()
            return carry

        lax.fori_loop(0, NCHUNK, chunk, 0)

    return sc_rot


def kernel(x, mem_subject, mem_object, mem_verb, mem_rot, W_role, b_role,
           W_spawn, b_spawn, W_subj, b_subj, W_obj, b_obj, W_verb_gate,
           b_verb_gate, W_verb_c, b_verb_c, W_rot, b_rot, W_out_gate,
           b_out_gate, ln_gamma, ln_beta, role_label, rot_ptr,
           subject_locked, object_locked):
    B, d = x.shape
    R = mem_rot.shape[1]
    nroles = W_role.shape[1]
    BB = 128

    def row_blk(i):
        return (i, 0)

    def full(i):
        return (0, 0)

    row_spec = pl.BlockSpec((BB, d), row_blk)
    mask_spec = pl.BlockSpec((BB, 1), row_blk)
    w_spec = pl.BlockSpec((d, d), full)
    vcol_spec = pl.BlockSpec((d, 1), full)
    brow_spec = pl.BlockSpec((1, d), full)
    scal_spec = pl.BlockSpec((1, 1), full)

    role2d = role_label.reshape(B, 1)
    ptr2d = rot_ptr.reshape(B, 1)

    BA = 512
    rva_spec = pl.BlockSpec((BA, d), row_blk)
    rv, sel = pl.pallas_call(
        _rotvec_body,
        grid=(B // BA,),
        in_specs=[rva_spec, w_spec, brow_spec,
                  pl.BlockSpec((BA, 1), row_blk), pl.BlockSpec((BA, 1), row_blk)],
        out_specs=(rva_spec, pl.BlockSpec((BA, 1), row_blk)),
        out_shape=(jax.ShapeDtypeStruct((B, d), jnp.float32),
                   jax.ShapeDtypeStruct((B, 1), jnp.int32)),
    )(x, W_rot, b_rot.reshape(1, d), role2d, ptr2d)

    zrows = jnp.zeros((R, d), jnp.float32)
    rot_flat = _sc_rot(B, R, d)(
        mem_rot.reshape(B * R, d), x, rv, sel.reshape(B), zrows)
    rot = rot_flat.reshape(B, R, d)

    out_shapes = (
        jax.ShapeDtypeStruct((B, d), jnp.float32),
        jax.ShapeDtypeStruct((B, nroles), jnp.float32),
        jax.ShapeDtypeStruct((B, 1), jnp.float32),
        jax.ShapeDtypeStruct((B, d), jnp.float32),
        jax.ShapeDtypeStruct((B, d), jnp.float32),
        jax.ShapeDtypeStruct((B, d), jnp.float32),
    )
    out_specs = (
        row_spec,
        pl.BlockSpec((BB, nroles), row_blk),
        mask_spec,
        row_spec,
        row_spec,
        row_spec,
    )
    in_specs = [
        row_spec, row_spec, row_spec, row_spec,
        pl.BlockSpec((d, nroles), full), pl.BlockSpec((1, nroles), full),
        vcol_spec, scal_spec,
        w_spec, brow_spec, w_spec, brow_spec,
        vcol_spec, scal_spec, w_spec, brow_spec,
        w_spec, brow_spec,
        brow_spec, brow_spec,
        mask_spec, mask_spec, mask_spec,
    ]
    x_out, role_logits, spawn_logit, new_s, new_o, new_v = pl.pallas_call(
        _main_body,
        grid=(B // BB,),
        in_specs=in_specs,
        out_specs=out_specs,
        out_shape=out_shapes,
    )(
        x, mem_subject, mem_object, mem_verb,
        W_role, b_role.reshape(1, nroles),
        W_spawn.reshape(d, 1), b_spawn.reshape(1, 1),
        W_subj, b_subj.reshape(1, d), W_obj, b_obj.reshape(1, d),
        W_verb_gate.reshape(d, 1), b_verb_gate.reshape(1, 1),
        W_verb_c, b_verb_c.reshape(1, d),
        W_out_gate, b_out_gate.reshape(1, d),
        ln_gamma.reshape(1, d), ln_beta.reshape(1, d),
        role2d,
        subject_locked.astype(jnp.int32).reshape(B, 1),
        object_locked.astype(jnp.int32).reshape(B, 1),
    )
    return (x_out, role_logits, spawn_logit.reshape(B), new_s, new_o,
            new_v, rot)
